# Initial kernel scaffold; baseline (speedup 1.0000x reference)
import jax
import jax.numpy as jnp
from jax.experimental import pallas as pl


def kernel(x, edge_index, batch, conv_w1, conv_b1, conv_w2, conv_b2, bn_g, bn_b, weight1, weight2, weight3, w_fc_xd, b_fc_xd, w_fc1, b_fc1, w_fc2, b_fc2, w_out, b_out):
    raise NotImplementedError("write your pallas kernel here")


# speedup vs baseline: 6.9998x; 6.9998x over previous
import functools

import jax
import jax.numpy as jnp
from jax import lax
from jax.experimental import pallas as pl
from jax.experimental.pallas import tpu as pltpu
from jax.experimental.pallas import tpu_sc as plsc

_N, _E, _D, _G = 10000, 320000, 128, 80
_NC, _NS = 2, 16
_NW = _NC * _NS
_CHUNK = 128
_CH = 79
_EPT = _CH * _CHUNK
_EPAD = _NW * _EPT
_NPAD = 10112
_RPT = _NPAD // _NS


def _sc_segsum_body(h_hbm, src_hbm, dst_hbm, zeros_hbm, out_hbm,
                    idx_s, idx_d, rows, acc, sem):
    c = lax.axis_index("c")
    s = lax.axis_index("s")
    wid = s * _NC + c
    pltpu.sync_copy(src_hbm.at[wid], idx_s)
    pltpu.sync_copy(dst_hbm.at[wid], idx_d)
    pltpu.sync_copy(zeros_hbm.at[pl.ds(s * _RPT, _RPT)],
                    acc.at[pl.ds(s * _RPT, _RPT)])
    plsc.subcore_barrier()

    def chunk(j, carry):
        pltpu.async_copy(h_hbm.at[idx_s.at[j]], rows, sem).wait()
        pltpu.sync_copy(rows, acc.at[idx_d.at[j]], add=True)
        return carry

    lax.fori_loop(0, _CH, chunk, 0)
    plsc.subcore_barrier()
    pltpu.sync_copy(acc.at[pl.ds(s * _RPT, _RPT)],
                    out_hbm.at[c, pl.ds(s * _RPT, _RPT)])


@functools.lru_cache(maxsize=None)
def _get_sc_segsum():
    return pl.kernel(
        _sc_segsum_body,
        out_type=jax.ShapeDtypeStruct((_NC, _NPAD, _D), jnp.float32),
        mesh=plsc.VectorSubcoreMesh(core_axis_name="c", subcore_axis_name="s",
                                    num_cores=_NC, num_subcores=_NS),
        scratch_types=[
            pltpu.VMEM((_CH, _CHUNK), jnp.int32),
            pltpu.VMEM((_CH, _CHUNK), jnp.int32),
            pltpu.VMEM((_CHUNK, _D), jnp.float32),
            pltpu.VMEM_SHARED((_NPAD, _D), jnp.float32),
            pltpu.SemaphoreType.DMA,
        ],
    )


def _dot(a, b):
    return jnp.dot(a, b, preferred_element_type=jnp.float32)


def _hp_dot(a, b):
    return jnp.dot(a, b, preferred_element_type=jnp.float32,
                   precision=lax.Precision.HIGHEST)


def _layer_math(h, agg0, agg1, w1, b1, w2, b2, g, bb):
    z = h + agg0 + agg1
    t = jnp.maximum(_dot(z, w1) + b1, 0.0)
    r = jnp.maximum(_dot(t, w2) + b2, 0.0)
    m = jnp.mean(r, axis=0, keepdims=True)
    v = jnp.mean((r - m) ** 2, axis=0, keepdims=True)
    return (r - m) / jnp.sqrt(v + 1e-5) * g + bb


def _tc_layer_body(mode, *refs):
    if mode == 0:
        (h_ref, agg_ref, w1_ref, b1_ref, w2_ref, b2_ref, g_ref, bb_ref,
         o_ref, ys_ref) = refs
    elif mode == 1:
        (h_ref, agg_ref, w1_ref, b1_ref, w2_ref, b2_ref, g_ref, bb_ref,
         ysin_ref, w_ref, o_ref, ys_ref) = refs
    else:
        (h_ref, agg_ref, w1_ref, b1_ref, w2_ref, b2_ref, g_ref, bb_ref,
         ysin_ref, o_ref, ys_ref) = refs
    hout = _layer_math(h_ref[...], agg_ref[0, :_N, :], agg_ref[1, :_N, :],
                       w1_ref[...], b1_ref[...], w2_ref[...], b2_ref[...],
                       g_ref[...], bb_ref[...])
    o_ref[...] = hout
    if mode == 0:
        ys_ref[...] = hout
    elif mode == 1:
        ys_ref[...] = ysin_ref[...] + _dot(hout, w_ref[...])
    else:
        ys_ref[...] = ysin_ref[...] + hout


def _tc_layer(mode, *args):
    return pl.pallas_call(
        functools.partial(_tc_layer_body, mode),
        out_shape=[jax.ShapeDtypeStruct((_N, _D), jnp.float32),
                   jax.ShapeDtypeStruct((_N, _D), jnp.float32)],
    )(*args)


def _tc_final_body(ys_ref, batch_ref, wxd_ref, bxd_ref,
                   wf1_ref, bf1_ref, wf2_ref, bf2_ref, wo_ref, bo_ref,
                   out_ref, pooled_ref):
    gids = lax.broadcasted_iota(jnp.int32, (_G, _N), 0)
    mask = (gids == batch_ref[...]).astype(jnp.float32)
    pooled = _hp_dot(mask, ys_ref[...])
    pooled_ref[...] = pooled
    xd = jnp.maximum(_dot(pooled, wxd_ref[...]) + bxd_ref[...], 0.0)
    xc = jnp.maximum(_dot(xd, wf1_ref[...]) + bf1_ref[...], 0.0)
    xc = jnp.maximum(_dot(xc, wf2_ref[...]) + bf2_ref[...], 0.0)
    out_ref[...] = _dot(xc, wo_ref[...]) + bo_ref[...]


def kernel(x, edge_index, batch, conv_w1, conv_b1, conv_w2, conv_b2,
           bn_g, bn_b, weight1, weight2, weight3, w_fc_xd, b_fc_xd,
           w_fc1, b_fc1, w_fc2, b_fc2, w_out, b_out):
    src = edge_index[0]
    dst = edge_index[1]
    npad = _EPAD - _E
    ar = jnp.arange(npad, dtype=jnp.int32)
    src_p = jnp.concatenate([src, (ar * 37) % _N]).reshape(_NW, _CH, _CHUNK)
    dst_p = jnp.concatenate([dst, _N + (ar % (_NPAD - _N))]
                            ).reshape(_NW, _CH, _CHUNK)
    zeros = jnp.zeros((_NPAD, _D), jnp.float32)

    gin_w = [weight1, weight2, weight3]
    h = x
    ys = None
    for i in range(5):
        agg = _get_sc_segsum()(h, src_p, dst_p, zeros)
        common = (h, agg, conv_w1[i], conv_b1[i].reshape(1, _D),
                  conv_w2[i], conv_b2[i].reshape(1, _D),
                  bn_g[i].reshape(1, _D), bn_b[i].reshape(1, _D))
        if i == 0:
            h, ys = _tc_layer(0, *common)
        elif i < 4:
            h, ys = _tc_layer(1, *common, ys, gin_w[i - 1])
        else:
            h, ys = _tc_layer(2, *common, ys)

    out, pooled = pl.pallas_call(
        _tc_final_body,
        out_shape=[jax.ShapeDtypeStruct((_G, 1), jnp.float32),
                   jax.ShapeDtypeStruct((_G, _D), jnp.float32)],
    )(ys, batch.reshape(1, _N),
      w_fc_xd, b_fc_xd.reshape(1, _D),
      w_fc1, b_fc1.reshape(1, 1024), w_fc2, b_fc2.reshape(1, 256),
      w_out, b_out.reshape(1, 1))
    return (out, pooled.reshape(_G, 1, _D))
